# baseline (speedup 1.0000x reference)
import functools

import jax
import jax.numpy as jnp
from jax import lax
from jax.experimental import pallas as pl
from jax.experimental.pallas import tpu as pltpu
from jax.experimental.pallas import tpu_sc as plsc

N = 10000
NE = 5000
NNZ = 320000
C = 128
L = 4
NCLS = 4
EPS = 1e-5

NW = 32
K = 128
CH = 79
NNZ_PAD = NW * CH * K
TCH = NNZ_PAD // K
CH0 = 150
CH1 = 8
NP = 10112
EP = 5120

_MESH = plsc.VectorSubcoreMesh(core_axis_name="c", subcore_axis_name="s")


def _seg_sum_body(nch, nbuf, table_hbm, g_hbm, d_hbm, out_hbm,
                  gt, dt, rows, gsem, ssem, isg, isd, acc):
    cid = lax.axis_index("c")
    sid = lax.axis_index("s")
    chc = jnp.where(cid == 0, CH0, CH1)
    base = cid * (16 * CH0) + sid * chc

    z16 = jnp.zeros((16,), jnp.float32)

    @pl.loop(0, K)
    def _fill(i):
        for c8 in range(C // 16):
            rows[0, i, pl.ds(c8 * 16, 16)] = z16

    @pl.loop(sid, nch, step=16)
    def _zero(cc):
        pltpu.sync_copy(rows.at[0], acc.at[pl.ds(cc * K, K)])

    for b in range(nbuf):
        pltpu.async_copy(g_hbm.at[pl.ds(base + b, 1)],
                         gt.at[pl.ds(b, 1)], isg.at[b])
        pltpu.async_copy(d_hbm.at[pl.ds(base + b, 1)],
                         dt.at[pl.ds(b, 1)], isd.at[b])
    for b in range(nbuf):
        pltpu.make_async_copy(g_hbm.at[pl.ds(base, 1)],
                              gt.at[pl.ds(b, 1)], isg.at[b]).wait()
        pltpu.async_copy(table_hbm.at[gt.at[b]], rows.at[b], gsem.at[b])

    plsc.subcore_barrier()

    def process(j, b):
        pltpu.make_async_copy(d_hbm.at[pl.ds(base, 1)],
                              dt.at[pl.ds(b, 1)], isd.at[b]).wait()
        pltpu.make_async_copy(table_hbm.at[gt.at[b]],
                              rows.at[b], gsem.at[b]).wait()
        pltpu.async_copy(rows.at[b], acc.at[dt.at[b]], ssem.at[b],
                         add=True)

        @pl.when(j + nbuf < chc)
        def _():
            pltpu.async_copy(g_hbm.at[pl.ds(base + j + nbuf, 1)],
                             gt.at[pl.ds(b, 1)], isg.at[b])
            pltpu.make_async_copy(rows.at[b], acc.at[dt.at[b]],
                                  ssem.at[b]).wait()
            pltpu.async_copy(d_hbm.at[pl.ds(base + j + nbuf, 1)],
                             dt.at[pl.ds(b, 1)], isd.at[b])
            pltpu.make_async_copy(g_hbm.at[pl.ds(base, 1)],
                                  gt.at[pl.ds(b, 1)], isg.at[b]).wait()
            pltpu.async_copy(table_hbm.at[gt.at[b]], rows.at[b], gsem.at[b])

    n_full = (chc // nbuf) * nbuf

    @pl.loop(0, n_full, step=nbuf)
    def _accum(j0):
        for b in range(nbuf):
            process(j0 + b, b)

    for b in range(nbuf - 1):
        @pl.when(n_full + b < chc)
        def _():
            process(n_full + b, b)
    for b in range(nbuf):
        pltpu.make_async_copy(rows.at[b], acc.at[dt.at[b]],
                              ssem.at[b]).wait()

    plsc.subcore_barrier()

    @pl.loop(sid, nch, step=16)
    def _flush(cc):
        pltpu.sync_copy(acc.at[pl.ds(cc * K, K)],
                        out_hbm.at[cid, pl.ds(cc * K, K)])


def _sc_segment_sum(table, gidx, didx, n_rows, nbuf):
    nch = n_rows // K
    return pl.kernel(
        functools.partial(_seg_sum_body, nch, nbuf),
        out_type=jax.ShapeDtypeStruct((2, n_rows, C), jnp.float32),
        mesh=_MESH,
        scratch_types=[
            pltpu.VMEM((nbuf, K), jnp.int32),
            pltpu.VMEM((nbuf, K), jnp.int32),
            pltpu.VMEM((nbuf, K, C), jnp.float32),
            pltpu.SemaphoreType.DMA((nbuf,)),
            pltpu.SemaphoreType.DMA((nbuf,)),
            pltpu.SemaphoreType.DMA((nbuf,)),
            pltpu.SemaphoreType.DMA((nbuf,)),
            pltpu.VMEM_SHARED((n_rows, C), jnp.float32),
        ],
    )(table, gidx, didx)


def _deg_body(nch, d_hbm, ones_hbm, out_hbm, dbuf, ones_v, zbuf, ssem, acc):
    cid = lax.axis_index("c")
    sid = lax.axis_index("s")
    wid = cid * 16 + sid
    pltpu.sync_copy(d_hbm.at[wid], dbuf)
    pltpu.sync_copy(ones_hbm, ones_v)
    z16 = jnp.zeros((16,), jnp.float32)

    @pl.loop(0, K)
    def _fill(i):
        for c8 in range(C // 16):
            zbuf[i, pl.ds(c8 * 16, 16)] = z16

    @pl.loop(sid, nch, step=16)
    def _zero(cc):
        pltpu.sync_copy(zbuf, acc.at[pl.ds(cc * K, K)])

    plsc.subcore_barrier()

    n_full = (CH // 4) * 4

    @pl.loop(0, n_full, step=4)
    def _accum(j0):
        for b in range(4):
            j = j0 + b

            @pl.when(j >= 4)
            def _():
                pltpu.make_async_copy(ones_v, acc.at[dbuf.at[0]],
                                      ssem.at[b]).wait()

            pltpu.async_copy(ones_v, acc.at[dbuf.at[j]], ssem.at[b],
                             add=True)

    for jt in range(n_full, CH):
        b = jt - n_full
        pltpu.make_async_copy(ones_v, acc.at[dbuf.at[0]], ssem.at[b]).wait()
        pltpu.async_copy(ones_v, acc.at[dbuf.at[jt]], ssem.at[b], add=True)
    for b in range(4):
        pltpu.make_async_copy(ones_v, acc.at[dbuf.at[0]], ssem.at[b]).wait()

    plsc.subcore_barrier()

    @pl.loop(sid, nch, step=16)
    def _flush(cc):
        pltpu.sync_copy(acc.at[pl.ds(cc * K, K)],
                        out_hbm.at[cid, pl.ds(cc * K, K)])


def _sc_degrees(didx, n_rows, ones_blk):
    nch = n_rows // K
    return pl.kernel(
        functools.partial(_deg_body, nch),
        out_type=jax.ShapeDtypeStruct((2, n_rows, C), jnp.float32),
        mesh=_MESH,
        scratch_types=[
            pltpu.VMEM((CH, K), jnp.int32),
            pltpu.VMEM((K, C), jnp.float32),
            pltpu.VMEM((K, C), jnp.float32),
            pltpu.SemaphoreType.DMA((4,)),
            pltpu.VMEM_SHARED((n_rows, C), jnp.float32),
        ],
    )(didx, ones_blk)


def _dense(h, g, be, w, b):
    m = jnp.mean(h, axis=1, keepdims=True)
    d = h - m
    v = jnp.mean(d * d, axis=1, keepdims=True)
    t = d * lax.rsqrt(v + EPS) * g + be
    t = jnp.maximum(t, 0.0)
    return jnp.dot(t, w, preferred_element_type=jnp.float32) + b


def _res(z_ref, dv_ref, h_ref):
    z = z_ref[0] + z_ref[1]
    d = dv_ref[0, :, 0:1] + dv_ref[1, :, 0:1]
    return h_ref[...] + jnp.maximum(z * (1.0 / jnp.maximum(d, 1.0)), 0.0)


def _enc_dense_body(x_ref, we_ref, bee_ref, g_ref, be_ref, w_ref, b_ref,
                    h_ref, t_ref):
    h = (jnp.dot(x_ref[...], we_ref[...],
                 preferred_element_type=jnp.float32) + bee_ref[...])
    h_ref[...] = h
    t_ref[...] = _dense(h, g_ref[...], be_ref[...], w_ref[...], b_ref[...])


def _combine_body(s_ref, de_ref, o_ref):
    s = s_ref[0] + s_ref[1]
    d = de_ref[0, :, 0:1] + de_ref[1, :, 0:1]
    o_ref[...] = s * (1.0 / jnp.maximum(d, 1.0))


def _crd_body(z_ref, dv_ref, h_ref, g_ref, be_ref, w_ref, b_ref,
              hn_ref, t_ref):
    hn = _res(z_ref, dv_ref, h_ref)
    hn_ref[...] = hn
    t_ref[...] = _dense(hn, g_ref[...], be_ref[...], w_ref[...], b_ref[...])


def _cr_logits_body(z_ref, dv_ref, h_ref, w_ref, b_ref, o_ref):
    hn = _res(z_ref, dv_ref, h_ref)
    lo = (jnp.dot(hn, w_ref[...],
                  preferred_element_type=jnp.float32) + b_ref[...])
    m = jnp.max(lo, axis=1, keepdims=True)
    p = lo - m
    lse = jnp.log(jnp.sum(jnp.exp(p), axis=1, keepdims=True))
    o_ref[...] = p - lse


def _tc(body, shapes, *args):
    if isinstance(shapes, tuple) and isinstance(shapes[0], tuple):
        out = tuple(jax.ShapeDtypeStruct(s, jnp.float32) for s in shapes)
    else:
        out = jax.ShapeDtypeStruct(shapes, jnp.float32)
    return pl.pallas_call(body, out_shape=out)(*args)


def kernel(X, v_idx, e_idx, W_enc, b_enc, Wt, bt, gamma, beta, W_lin, b_lin):
    f32 = jnp.float32
    i32 = jnp.int32
    pad = NNZ_PAD - NNZ
    shape2 = (TCH, K)
    vg = jnp.concatenate([v_idx, jnp.zeros((pad,), i32)]).reshape(shape2)
    eg = jnp.concatenate([e_idx, jnp.zeros((pad,), i32)]).reshape(shape2)
    ed = jnp.concatenate([e_idx, jnp.full((pad,), NE, i32)]).reshape(shape2)
    vd = jnp.concatenate([v_idx, jnp.full((pad,), N, i32)]).reshape(shape2)
    ed3 = ed.reshape(NW, CH, K)
    vd3 = vd.reshape(NW, CH, K)
    Xp = jnp.zeros((NP, C), f32).at[:N].set(X)
    ones_blk = jnp.ones((K, C), f32)

    de = _sc_degrees(ed3, EP, ones_blk)
    dv = _sc_degrees(vd3, NP, ones_blk)
    h, t = _tc(_enc_dense_body, ((NP, C), (NP, C)), Xp, W_enc,
               b_enc.reshape(1, C), gamma[0].reshape(1, C),
               beta[0].reshape(1, C), Wt[0], bt[0].reshape(1, C))
    Wl = jnp.zeros((C, C), f32).at[:, :NCLS].set(W_lin)
    bl = jnp.full((1, C), -1e30, f32).at[0, :NCLS].set(b_lin)
    for l in range(L):
        s = _sc_segment_sum(t, vg, ed, EP, nbuf=4)
        y = _tc(_combine_body, (EP, C), s, de)
        z = _sc_segment_sum(y, eg, vd, NP, nbuf=3)
        if l < L - 1:
            h, t = _tc(_crd_body, ((NP, C), (NP, C)), z, dv, h,
                       gamma[l + 1].reshape(1, C), beta[l + 1].reshape(1, C),
                       Wt[l + 1], bt[l + 1].reshape(1, C))
        else:
            lg = _tc(_cr_logits_body, (NP, C), z, dv, h, Wl, bl)
    return lg[:N, :NCLS]


# speedup vs baseline: 1.0238x; 1.0238x over previous
import functools

import jax
import jax.numpy as jnp
from jax import lax
from jax.experimental import pallas as pl
from jax.experimental.pallas import tpu as pltpu
from jax.experimental.pallas import tpu_sc as plsc

N = 10000
NE = 5000
NNZ = 320000
C = 128
L = 4
NCLS = 4
EPS = 1e-5

NW = 32
K = 128
CH = 79
NNZ_PAD = NW * CH * K
TCH = NNZ_PAD // K
CH0 = 150
CH1 = 8
NP = 10112
EP = 5120

_MESH = plsc.VectorSubcoreMesh(core_axis_name="c", subcore_axis_name="s")


def _seg_sum_body(nch, nbuf, table_hbm, g_hbm, d_hbm, out_hbm,
                  gt, dt, rows, gsem, ssem, isg, isd, acc):
    cid = lax.axis_index("c")
    sid = lax.axis_index("s")
    chc = jnp.where(cid == 0, CH0, CH1)
    base = cid * (16 * CH0) + sid * chc

    z16 = jnp.zeros((16,), jnp.float32)

    @pl.loop(0, K)
    def _fill(i):
        for c8 in range(C // 16):
            rows[0, i, pl.ds(c8 * 16, 16)] = z16

    @pl.loop(sid, nch, step=16)
    def _zero(cc):
        pltpu.sync_copy(rows.at[0], acc.at[pl.ds(cc * K, K)])

    for b in range(nbuf):
        pltpu.async_copy(g_hbm.at[pl.ds(base + b, 1)],
                         gt.at[pl.ds(b, 1)], isg.at[b])
        pltpu.async_copy(d_hbm.at[pl.ds(base + b, 1)],
                         dt.at[pl.ds(b, 1)], isd.at[b])
    for b in range(nbuf):
        pltpu.make_async_copy(g_hbm.at[pl.ds(base, 1)],
                              gt.at[pl.ds(b, 1)], isg.at[b]).wait()
        pltpu.async_copy(table_hbm.at[gt.at[b]], rows.at[b], gsem.at[b])

    plsc.subcore_barrier()

    def process(j, b):
        pltpu.make_async_copy(d_hbm.at[pl.ds(base, 1)],
                              dt.at[pl.ds(b, 1)], isd.at[b]).wait()
        pltpu.make_async_copy(table_hbm.at[gt.at[b]],
                              rows.at[b], gsem.at[b]).wait()
        pltpu.async_copy(rows.at[b], acc.at[dt.at[b]], ssem.at[b],
                         add=True)

        @pl.when(j + nbuf < chc)
        def _():
            pltpu.async_copy(g_hbm.at[pl.ds(base + j + nbuf, 1)],
                             gt.at[pl.ds(b, 1)], isg.at[b])
            pltpu.make_async_copy(rows.at[b], acc.at[dt.at[b]],
                                  ssem.at[b]).wait()
            pltpu.async_copy(d_hbm.at[pl.ds(base + j + nbuf, 1)],
                             dt.at[pl.ds(b, 1)], isd.at[b])
            pltpu.make_async_copy(g_hbm.at[pl.ds(base, 1)],
                                  gt.at[pl.ds(b, 1)], isg.at[b]).wait()
            pltpu.async_copy(table_hbm.at[gt.at[b]], rows.at[b], gsem.at[b])

    n_full = (chc // nbuf) * nbuf

    @pl.loop(0, n_full, step=nbuf)
    def _accum(j0):
        for b in range(nbuf):
            process(j0 + b, b)

    for b in range(nbuf - 1):
        @pl.when(n_full + b < chc)
        def _():
            process(n_full + b, b)
    for b in range(nbuf):
        pltpu.make_async_copy(rows.at[b], acc.at[dt.at[b]],
                              ssem.at[b]).wait()

    plsc.subcore_barrier()

    @pl.loop(sid, nch, step=16)
    def _flush(cc):
        pltpu.sync_copy(acc.at[pl.ds(cc * K, K)],
                        out_hbm.at[cid, pl.ds(cc * K, K)])


def _sc_segment_sum(table, gidx, didx, n_rows, nbuf):
    nch = n_rows // K
    return pl.kernel(
        functools.partial(_seg_sum_body, nch, nbuf),
        out_type=jax.ShapeDtypeStruct((2, n_rows, C), jnp.float32),
        mesh=_MESH,
        scratch_types=[
            pltpu.VMEM((nbuf, K), jnp.int32),
            pltpu.VMEM((nbuf, K), jnp.int32),
            pltpu.VMEM((nbuf, K, C), jnp.float32),
            pltpu.SemaphoreType.DMA((nbuf,)),
            pltpu.SemaphoreType.DMA((nbuf,)),
            pltpu.SemaphoreType.DMA((nbuf,)),
            pltpu.SemaphoreType.DMA((nbuf,)),
            pltpu.VMEM_SHARED((n_rows, C), jnp.float32),
        ],
    )(table, gidx, didx)


def _deg_body(nch, d_hbm, ones_hbm, out_hbm, dbuf, ones_v, zbuf, ssem, acc):
    cid = lax.axis_index("c")
    sid = lax.axis_index("s")
    wid = cid * 16 + sid
    pltpu.sync_copy(d_hbm.at[wid], dbuf)
    pltpu.sync_copy(ones_hbm, ones_v)
    z16 = jnp.zeros((16,), jnp.float32)

    @pl.loop(0, K)
    def _fill(i):
        for c8 in range(C // 16):
            zbuf[i, pl.ds(c8 * 16, 16)] = z16

    @pl.loop(sid, nch, step=16)
    def _zero(cc):
        pltpu.sync_copy(zbuf, acc.at[pl.ds(cc * K, K)])

    plsc.subcore_barrier()

    n_full = (CH // 4) * 4

    @pl.loop(0, n_full, step=4)
    def _accum(j0):
        for b in range(4):
            j = j0 + b

            @pl.when(j >= 4)
            def _():
                pltpu.make_async_copy(ones_v, acc.at[dbuf.at[0]],
                                      ssem.at[b]).wait()

            pltpu.async_copy(ones_v, acc.at[dbuf.at[j]], ssem.at[b],
                             add=True)

    for jt in range(n_full, CH):
        b = jt - n_full
        pltpu.make_async_copy(ones_v, acc.at[dbuf.at[0]], ssem.at[b]).wait()
        pltpu.async_copy(ones_v, acc.at[dbuf.at[jt]], ssem.at[b], add=True)
    for b in range(4):
        pltpu.make_async_copy(ones_v, acc.at[dbuf.at[0]], ssem.at[b]).wait()

    plsc.subcore_barrier()

    @pl.loop(sid, nch, step=16)
    def _flush(cc):
        pltpu.sync_copy(acc.at[pl.ds(cc * K, K)],
                        out_hbm.at[cid, pl.ds(cc * K, K)])


def _sc_degrees(didx, n_rows, ones_blk):
    nch = n_rows // K
    return pl.kernel(
        functools.partial(_deg_body, nch),
        out_type=jax.ShapeDtypeStruct((2, n_rows, C), jnp.float32),
        mesh=_MESH,
        scratch_types=[
            pltpu.VMEM((CH, K), jnp.int32),
            pltpu.VMEM((K, C), jnp.float32),
            pltpu.VMEM((K, C), jnp.float32),
            pltpu.SemaphoreType.DMA((4,)),
            pltpu.VMEM_SHARED((n_rows, C), jnp.float32),
        ],
    )(didx, ones_blk)


def _dense(h, g, be, w, b):
    m = jnp.mean(h, axis=1, keepdims=True)
    d = h - m
    v = jnp.mean(d * d, axis=1, keepdims=True)
    t = d * lax.rsqrt(v + EPS) * g + be
    t = jnp.maximum(t, 0.0)
    return jnp.dot(t, w, preferred_element_type=jnp.float32) + b


def _res(z_ref, dv_ref, h_ref):
    z = z_ref[0] + z_ref[1]
    d = dv_ref[0, :, 0:1] + dv_ref[1, :, 0:1]
    return h_ref[...] + jnp.maximum(z * (1.0 / jnp.maximum(d, 1.0)), 0.0)


def _enc_body(x_ref, w_ref, b_ref, o_ref):
    o_ref[...] = (jnp.dot(x_ref[...], w_ref[...],
                          preferred_element_type=jnp.float32) + b_ref[...])


def _dense_body(h_ref, g_ref, be_ref, w_ref, b_ref, o_ref):
    o_ref[...] = _dense(h_ref[...], g_ref[...], be_ref[...],
                        w_ref[...], b_ref[...])


def _combine_body(s_ref, de_ref, o_ref):
    s = s_ref[0] + s_ref[1]
    d = de_ref[0, :, 0:1] + de_ref[1, :, 0:1]
    o_ref[...] = s * (1.0 / jnp.maximum(d, 1.0))


def _combine_res_body(z_ref, dv_ref, h_ref, o_ref):
    o_ref[...] = _res(z_ref, dv_ref, h_ref)


def _logits_body(h_ref, w_ref, b_ref, o_ref):
    lo = (jnp.dot(h_ref[...], w_ref[...],
                  preferred_element_type=jnp.float32) + b_ref[...])
    m = jnp.max(lo, axis=1, keepdims=True)
    p = lo - m
    lse = jnp.log(jnp.sum(jnp.exp(p), axis=1, keepdims=True))
    o_ref[...] = p - lse


def _tc(body, shapes, *args):
    if isinstance(shapes, tuple) and isinstance(shapes[0], tuple):
        out = tuple(jax.ShapeDtypeStruct(s, jnp.float32) for s in shapes)
    else:
        out = jax.ShapeDtypeStruct(shapes, jnp.float32)
    return pl.pallas_call(body, out_shape=out)(*args)


def kernel(X, v_idx, e_idx, W_enc, b_enc, Wt, bt, gamma, beta, W_lin, b_lin):
    f32 = jnp.float32
    i32 = jnp.int32
    pad = NNZ_PAD - NNZ
    shape2 = (TCH, K)
    vg = jnp.concatenate([v_idx, jnp.zeros((pad,), i32)]).reshape(shape2)
    eg = jnp.concatenate([e_idx, jnp.zeros((pad,), i32)]).reshape(shape2)
    ed = jnp.concatenate([e_idx, jnp.full((pad,), NE, i32)]).reshape(shape2)
    vd = jnp.concatenate([v_idx, jnp.full((pad,), N, i32)]).reshape(shape2)
    ed3 = ed.reshape(NW, CH, K)
    vd3 = vd.reshape(NW, CH, K)
    Xp = jnp.zeros((NP, C), f32).at[:N].set(X)
    ones_blk = jnp.ones((K, C), f32)

    de = _sc_degrees(ed3, EP, ones_blk)
    dv = _sc_degrees(vd3, NP, ones_blk)
    h = _tc(_enc_body, (NP, C), Xp, W_enc, b_enc.reshape(1, C))
    for l in range(L):
        t = _tc(_dense_body, (NP, C), h, gamma[l].reshape(1, C),
                beta[l].reshape(1, C), Wt[l], bt[l].reshape(1, C))
        s = _sc_segment_sum(t, vg, ed, EP, nbuf=4)
        y = _tc(_combine_body, (EP, C), s, de)
        z = _sc_segment_sum(y, eg, vd, NP, nbuf=3)
        h = _tc(_combine_res_body, (NP, C), z, dv, h)
    Wl = jnp.zeros((C, C), f32).at[:, :NCLS].set(W_lin)
    bl = jnp.full((1, C), -1e30, f32).at[0, :NCLS].set(b_lin)
    lg = _tc(_logits_body, (NP, C), h, Wl, bl)
    return lg[:N, :NCLS]
